# baseline (speedup 1.0000x reference)
import jax
import jax.numpy as jnp
from jax import lax
from jax.experimental import pallas as pl
from jax.experimental.pallas import tpu as pltpu
from jax.experimental.pallas import tpu_sc as plsc

_N = 10000
_E = 160000
_D = 384
_H = 128
_G = 64
_NP = 10240
_EPAD = 163840
_NT = 32
_NC0 = 48
_NC1 = 112
_NCM = 112
_CK = 64
_RPS = _NP // 16
_BM = 1024


def _sc_mesh():
    return plsc.VectorSubcoreMesh(core_axis_name="c", subcore_axis_name="s")


_SC_PARAMS = pltpu.CompilerParams(needs_layout_passes=False)


def _deg_body(col_r, w_r, degs, colb, wb, zv, stage, dacc):
    c = lax.axis_index("c")
    s = lax.axis_index("s")
    wid = c * 16 + s
    pltpu.sync_copy(col_r.at[wid], colb)
    pltpu.sync_copy(w_r.at[wid], wb)

    def z(i, _):
        zv[pl.ds(i * 16, 16)] = jnp.zeros((16,), jnp.float32)
        return 0
    lax.fori_loop(0, _RPS // 16, z, 0)
    pltpu.sync_copy(zv, dacc.at[pl.ds(s * _RPS, _RPS)])
    plsc.subcore_barrier()

    def step(ci, _):
        pltpu.sync_copy(wb.at[ci], dacc.at[colb.at[ci]], add=True)
        return 0
    lax.fori_loop(0, _NCM, step, 0)
    plsc.subcore_barrier()

    pltpu.sync_copy(dacc.at[pl.ds(s * _RPS, _RPS)], stage)
    pltpu.sync_copy(stage, degs.at[c, pl.ds(s * _RPS, _RPS)])


def _deg(col_r, w_r):
    return pl.kernel(
        _deg_body,
        out_type=jax.ShapeDtypeStruct((2, _NP), jnp.float32),
        mesh=_sc_mesh(),
        scratch_types=[
            pltpu.VMEM((_NCM, _CK), jnp.int32),
            pltpu.VMEM((_NCM, _CK), jnp.float32),
            pltpu.VMEM((_RPS,), jnp.float32),
            pltpu.VMEM((_RPS,), jnp.float32),
            pltpu.VMEM_SHARED((_NP,), jnp.float32),
        ],
        compiler_params=_SC_PARAMS,
    )(col_r, w_r)


def _norm_body(row_r, col_r, w_r, dinv, outn, outrc, rowb, colb, wb, dv):
    c = lax.axis_index("c")
    s = lax.axis_index("s")
    wid = c * 16 + s
    pltpu.sync_copy(row_r.at[wid], rowb)
    pltpu.sync_copy(col_r.at[wid], colb)
    pltpu.sync_copy(w_r.at[wid], wb)
    pltpu.sync_copy(dinv.at[0], dv)

    def ngroup(g, _):
        ci = g // (_CK // 16)
        off = (g % (_CK // 16)) * 16
        sl = pl.ds(off, 16)
        rv = rowb[ci, sl]
        cv = colb[ci, sl]
        wv = wb[ci, sl]
        wb[ci, sl] = (
            plsc.load_gather(dv, [rv]) * wv * plsc.load_gather(dv, [cv]))
        rowb[ci, sl] = rv + cv * 16384
        return 0
    lax.fori_loop(0, _NCM * (_CK // 16), ngroup, 0)
    pltpu.sync_copy(wb, outn.at[wid])
    pltpu.sync_copy(rowb, outrc.at[wid])


def _normk(row_r, col_r, w_r, dinv):
    return pl.kernel(
        _norm_body,
        out_type=[jax.ShapeDtypeStruct((_NT, _NCM, _CK), jnp.float32),
                  jax.ShapeDtypeStruct((_NT, _NCM, _CK), jnp.int32)],
        mesh=_sc_mesh(),
        scratch_types=[
            pltpu.VMEM((_NCM, _CK), jnp.int32),
            pltpu.VMEM((_NCM, _CK), jnp.int32),
            pltpu.VMEM((_NCM, _CK), jnp.float32),
            pltpu.VMEM((_NP,), jnp.float32),
        ],
        compiler_params=_SC_PARAMS,
    )(row_r, col_r, w_r, dinv)


def _msg_body(xw, rc_r, norm_r, out, rcb, nb, idxr,
              g0, g1, acc, sg0, sg1, ss0, ss1):
    c = lax.axis_index("c")
    s = lax.axis_index("s")
    wid = c * 16 + s
    pltpu.sync_copy(rc_r.at[wid], rcb)
    pltpu.sync_copy(norm_r.at[wid], nb)
    nch = jnp.where(c == 0, _NC0, _NC1)

    def z(i, _):
        for j in range(_H // 16):
            g0[i, pl.ds(j * 16, 16)] = jnp.zeros((16,), jnp.float32)
        return 0
    lax.fori_loop(0, _CK, z, 0)

    def zrow(k, _):
        pltpu.sync_copy(g0, acc.at[pl.ds(s * _RPS + k * _CK, _CK)])
        return 0
    lax.fori_loop(0, _RPS // _CK, zrow, 0)
    plsc.subcore_barrier()

    def unpack(ci, slot):
        def ug(g, _):
            sl = pl.ds(g * 16, 16)
            v = rcb[ci, sl]
            cv = lax.shift_right_logical(v, 14)
            idxr[slot, sl] = v - cv * 16384
            idxr[slot + 2, sl] = cv
            return 0
        lax.fori_loop(0, _CK // 16, ug, 0)

    def gather(slot, gb, sg):
        pltpu.async_copy(xw.at[idxr.at[slot]], gb, sg)

    def scale(ci, gb):
        def group(g, _):
            nv = nb[ci, pl.ds(g * 16, 16)]
            base = g * 16
            for j in range(16):
                nbr = jnp.full((16,), nv[j])
                for k in range(_H // 16):
                    sl = pl.ds(k * 16, 16)
                    gb[base + j, sl] = gb[base + j, sl] * nbr
            return 0
        lax.fori_loop(0, _CK // 16, group, 0)

    def scatter(slot, gb, ss):
        pltpu.async_copy(gb, acc.at[idxr.at[slot + 2]], ss, add=True)

    def wait_dma(gb, sem):
        pltpu.make_async_copy(xw.at[pl.ds(0, _CK)], gb, sem).wait()

    unpack(0, 0)
    gather(0, g0, sg0)
    unpack(1, 1)
    gather(1, g1, sg1)

    def pair(p, _):
        c0 = 2 * p
        c1 = c0 + 1
        wait_dma(g0, sg0)
        scale(c0, g0)
        scatter(0, g0, ss0)

        wait_dma(g1, sg1)
        scale(c1, g1)
        scatter(1, g1, ss1)

        wait_dma(g0, ss0)

        @pl.when(c0 + 2 < nch)
        def _():
            unpack(c0 + 2, 0)
            gather(0, g0, sg0)

        wait_dma(g1, ss1)

        @pl.when(c1 + 2 < nch)
        def _():
            unpack(c1 + 2, 1)
            gather(1, g1, sg1)
        return 0
    lax.fori_loop(0, nch // 2, pair, 0)
    plsc.subcore_barrier()

    def wout(k, _):
        pltpu.sync_copy(acc.at[pl.ds(s * _RPS + k * _CK, _CK)], g0)
        pltpu.sync_copy(g0, out.at[c, pl.ds(s * _RPS + k * _CK, _CK)])
        return 0
    lax.fori_loop(0, _RPS // _CK, wout, 0)


def _msg(xw, rc_r, norm_r):
    return pl.kernel(
        _msg_body,
        out_type=jax.ShapeDtypeStruct((2, _NP, _H), jnp.float32),
        mesh=_sc_mesh(),
        scratch_types=[
            pltpu.VMEM((_NCM, _CK), jnp.int32),
            pltpu.VMEM((_NCM, _CK), jnp.float32),
            pltpu.VMEM((4, _CK), jnp.int32),
            pltpu.VMEM((_CK, _H), jnp.float32),
            pltpu.VMEM((_CK, _H), jnp.float32),
            pltpu.VMEM_SHARED((_NP, _H), jnp.float32),
            pltpu.SemaphoreType.DMA,
            pltpu.SemaphoreType.DMA,
            pltpu.SemaphoreType.DMA,
            pltpu.SemaphoreType.DMA,
        ],
        compiler_params=_SC_PARAMS,
    )(xw, rc_r, norm_r)


def _dinv_body(degs, dinv, dinv2):
    d = degs[0:1, :] + degs[1:2, :] + 1.0
    r = lax.rsqrt(d)
    dinv[...] = r
    dinv2[...] = r * r


def _dinv_call(degs):
    return pl.pallas_call(
        _dinv_body,
        out_shape=[jax.ShapeDtypeStruct((1, _NP), jnp.float32)] * 2,
    )(degs)


def _mm1_body(x, w, o):
    o[...] = lax.dot_general(x[...], w[...], (((1,), (1,)), ((), ())),
                             preferred_element_type=jnp.float32)


def _mm1(x_p, w):
    return pl.pallas_call(
        _mm1_body,
        grid=(_NP // _BM,),
        in_specs=[
            pl.BlockSpec((_BM, _D), lambda i: (i, 0)),
            pl.BlockSpec((_H, _D), lambda i: (0, 0)),
        ],
        out_specs=pl.BlockSpec((_BM, _H), lambda i: (i, 0)),
        out_shape=jax.ShapeDtypeStruct((_NP, _H), jnp.float32),
    )(x_p, w)


def _fused_body(a0, a1, xw, d2, b, w, o):
    h = jnp.maximum(a0[...] + a1[...] + d2[...] * xw[...] + b[...], 0.0)
    o[...] = lax.dot_general(h, w[...], (((1,), (1,)), ((), ())),
                             preferred_element_type=jnp.float32)


def _fused(a0, a1, xw, d2c, br, w):
    return pl.pallas_call(
        _fused_body,
        grid=(_NP // _BM,),
        in_specs=[
            pl.BlockSpec((_BM, _H), lambda i: (i, 0)),
            pl.BlockSpec((_BM, _H), lambda i: (i, 0)),
            pl.BlockSpec((_BM, _H), lambda i: (i, 0)),
            pl.BlockSpec((_BM, 1), lambda i: (i, 0)),
            pl.BlockSpec((1, _H), lambda i: (0, 0)),
            pl.BlockSpec((_H, _H), lambda i: (0, 0)),
        ],
        out_specs=pl.BlockSpec((_BM, _H), lambda i: (i, 0)),
        out_shape=jax.ShapeDtypeStruct((_NP, _H), jnp.float32),
    )(a0, a1, xw, d2c, br, w)


def _final_body(a0, a1, xw, d2, b, batch, wl, bl, o, sums, cnt):
    i = pl.program_id(0)

    @pl.when(i == 0)
    def _():
        sums[...] = jnp.zeros_like(sums)
        cnt[...] = jnp.zeros_like(cnt)

    h = a0[...] + a1[...] + d2[...] * xw[...] + b[...]
    gi = lax.broadcasted_iota(jnp.int32, (_G, _BM), 0)
    m = (gi == batch[...]).astype(jnp.float32)
    sums[...] += lax.dot_general(m, h, (((1,), (0,)), ((), ())),
                                 preferred_element_type=jnp.float32)
    cnt[...] += jnp.sum(m, axis=1, keepdims=True)

    @pl.when(i == _NP // _BM - 1)
    def _():
        mean = sums[...] / jnp.maximum(cnt[...], 1.0)
        o[...] = lax.dot_general(mean, wl[...], (((1,), (1,)), ((), ())),
                                 preferred_element_type=jnp.float32) + bl[...]


def _final(a0, a1, xw, d2c, br, batch_p, wl, blr):
    return pl.pallas_call(
        _final_body,
        grid=(_NP // _BM,),
        in_specs=[
            pl.BlockSpec((_BM, _H), lambda i: (i, 0)),
            pl.BlockSpec((_BM, _H), lambda i: (i, 0)),
            pl.BlockSpec((_BM, _H), lambda i: (i, 0)),
            pl.BlockSpec((_BM, 1), lambda i: (i, 0)),
            pl.BlockSpec((1, _H), lambda i: (0, 0)),
            pl.BlockSpec((1, _BM), lambda i: (0, i)),
            pl.BlockSpec((2, _H), lambda i: (0, 0)),
            pl.BlockSpec((1, 2), lambda i: (0, 0)),
        ],
        out_specs=pl.BlockSpec((_G, 2), lambda i: (0, 0)),
        out_shape=jax.ShapeDtypeStruct((_G, 2), jnp.float32),
        scratch_shapes=[
            pltpu.VMEM((_G, _H), jnp.float32),
            pltpu.VMEM((_G, 1), jnp.float32),
        ],
    )(a0, a1, xw, d2c, br, batch_p, wl, blr)


def kernel(x, edge_index, edge_attr, batch, W1, b1, W2, b2, W3, b3, Wl, bl):
    pe = _EPAD - _E

    def _part(a):
        a0 = a[: 16 * _NC0 * _CK].reshape(16, _NC0, _CK)
        a1 = a[16 * _NC0 * _CK:].reshape(16, _NC1, _CK)
        o = jnp.zeros((_NT, _NCM, _CK), a.dtype)
        return o.at[:16, :_NC0].set(a0).at[16:, :_NC1].set(a1)

    row_r = _part(jnp.pad(edge_index[0], (0, pe)))
    col_r = _part(jnp.pad(edge_index[1], (0, pe)))
    w_r = _part(jnp.pad(edge_attr, (0, pe)))
    x_p = jnp.pad(x, ((0, _NP - _N), (0, 0)))
    batch_p = jnp.pad(batch, (0, _NP - _N), constant_values=_G).reshape(1, _NP)
    b1r = b1.reshape(1, _H)
    b2r = b2.reshape(1, _H)
    b3r = b3.reshape(1, _H)
    blr = bl.reshape(1, 2)

    degs = _deg(col_r, w_r)
    dinv, dinv2 = _dinv_call(degs)
    d2c = dinv2.reshape(_NP, 1)
    norm_r, rc_r = _normk(row_r, col_r, w_r, dinv)

    xw1 = _mm1(x_p, W1)
    acc = _msg(xw1, rc_r, norm_r)
    xw2 = _fused(acc[0], acc[1], xw1, d2c, b1r, W2)
    acc = _msg(xw2, rc_r, norm_r)
    xw3 = _fused(acc[0], acc[1], xw2, d2c, b2r, W3)
    acc = _msg(xw3, rc_r, norm_r)
    return _final(acc[0], acc[1], xw3, d2c, b3r, batch_p, Wl, blr)


# speedup vs baseline: 1.0571x; 1.0571x over previous
import jax
import jax.numpy as jnp
from jax import lax
from jax.experimental import pallas as pl
from jax.experimental.pallas import tpu as pltpu
from jax.experimental.pallas import tpu_sc as plsc

_N = 10000
_E = 160000
_D = 384
_H = 128
_G = 64
_NP = 10240
_EPAD = 163840
_NT = 32
_NC0 = 80
_NC1 = 80
_NCM = 80
_CK = 64
_RPS = _NP // 16
_BM = 1024


def _sc_mesh():
    return plsc.VectorSubcoreMesh(core_axis_name="c", subcore_axis_name="s")


_SC_PARAMS = pltpu.CompilerParams(needs_layout_passes=False)


def _deg_body(col_r, w_r, degs, colb, wb, zv, stage, dacc):
    c = lax.axis_index("c")
    s = lax.axis_index("s")
    wid = c * 16 + s
    pltpu.sync_copy(col_r.at[wid], colb)
    pltpu.sync_copy(w_r.at[wid], wb)

    def z(i, _):
        zv[pl.ds(i * 16, 16)] = jnp.zeros((16,), jnp.float32)
        return 0
    lax.fori_loop(0, _RPS // 16, z, 0)
    pltpu.sync_copy(zv, dacc.at[pl.ds(s * _RPS, _RPS)])
    plsc.subcore_barrier()

    def step(ci, _):
        pltpu.sync_copy(wb.at[ci], dacc.at[colb.at[ci]], add=True)
        return 0
    lax.fori_loop(0, _NCM, step, 0)
    plsc.subcore_barrier()

    pltpu.sync_copy(dacc.at[pl.ds(s * _RPS, _RPS)], stage)
    pltpu.sync_copy(stage, degs.at[c, pl.ds(s * _RPS, _RPS)])


def _deg(col_r, w_r):
    return pl.kernel(
        _deg_body,
        out_type=jax.ShapeDtypeStruct((2, _NP), jnp.float32),
        mesh=_sc_mesh(),
        scratch_types=[
            pltpu.VMEM((_NCM, _CK), jnp.int32),
            pltpu.VMEM((_NCM, _CK), jnp.float32),
            pltpu.VMEM((_RPS,), jnp.float32),
            pltpu.VMEM((_RPS,), jnp.float32),
            pltpu.VMEM_SHARED((_NP,), jnp.float32),
        ],
        compiler_params=_SC_PARAMS,
    )(col_r, w_r)


def _norm_body(row_r, col_r, w_r, dinv, outn, outrc, rowb, colb, wb, dv):
    c = lax.axis_index("c")
    s = lax.axis_index("s")
    wid = c * 16 + s
    pltpu.sync_copy(row_r.at[wid], rowb)
    pltpu.sync_copy(col_r.at[wid], colb)
    pltpu.sync_copy(w_r.at[wid], wb)
    pltpu.sync_copy(dinv.at[0], dv)

    def ngroup(g, _):
        ci = g // (_CK // 16)
        off = (g % (_CK // 16)) * 16
        sl = pl.ds(off, 16)
        rv = rowb[ci, sl]
        cv = colb[ci, sl]
        wv = wb[ci, sl]
        wb[ci, sl] = (
            plsc.load_gather(dv, [rv]) * wv * plsc.load_gather(dv, [cv]))
        rowb[ci, sl] = rv + cv * 16384
        return 0
    lax.fori_loop(0, _NCM * (_CK // 16), ngroup, 0)
    pltpu.sync_copy(wb, outn.at[wid])
    pltpu.sync_copy(rowb, outrc.at[wid])


def _normk(row_r, col_r, w_r, dinv):
    return pl.kernel(
        _norm_body,
        out_type=[jax.ShapeDtypeStruct((_NT, _NCM, _CK), jnp.float32),
                  jax.ShapeDtypeStruct((_NT, _NCM, _CK), jnp.int32)],
        mesh=_sc_mesh(),
        scratch_types=[
            pltpu.VMEM((_NCM, _CK), jnp.int32),
            pltpu.VMEM((_NCM, _CK), jnp.int32),
            pltpu.VMEM((_NCM, _CK), jnp.float32),
            pltpu.VMEM((_NP,), jnp.float32),
        ],
        compiler_params=_SC_PARAMS,
    )(row_r, col_r, w_r, dinv)


def _msg_body(xw, rc_r, norm_r, out, rcb, nb, idxr,
              g0, g1, acc, sg0, sg1, ss0, ss1):
    c = lax.axis_index("c")
    s = lax.axis_index("s")
    wid = c * 16 + s
    pltpu.sync_copy(rc_r.at[wid], rcb)
    pltpu.sync_copy(norm_r.at[wid], nb)
    nch = jnp.where(c == 0, _NC0, _NC1)

    def z(i, _):
        for j in range(_H // 16):
            g0[i, pl.ds(j * 16, 16)] = jnp.zeros((16,), jnp.float32)
        return 0
    lax.fori_loop(0, _CK, z, 0)

    def zrow(k, _):
        pltpu.sync_copy(g0, acc.at[pl.ds(s * _RPS + k * _CK, _CK)])
        return 0
    lax.fori_loop(0, _RPS // _CK, zrow, 0)
    plsc.subcore_barrier()

    def unpack(ci, slot):
        def ug(g, _):
            sl = pl.ds(g * 16, 16)
            v = rcb[ci, sl]
            cv = lax.shift_right_logical(v, 14)
            idxr[slot, sl] = v - cv * 16384
            idxr[slot + 2, sl] = cv
            return 0
        lax.fori_loop(0, _CK // 16, ug, 0)

    def gather(slot, gb, sg):
        pltpu.async_copy(xw.at[idxr.at[slot]], gb, sg)

    def scale(ci, gb):
        def group(g, _):
            nv = nb[ci, pl.ds(g * 16, 16)]
            base = g * 16
            for j in range(16):
                nbr = jnp.full((16,), nv[j])
                for k in range(_H // 16):
                    sl = pl.ds(k * 16, 16)
                    gb[base + j, sl] = gb[base + j, sl] * nbr
            return 0
        lax.fori_loop(0, _CK // 16, group, 0)

    def scatter(slot, gb, ss):
        pltpu.async_copy(gb, acc.at[idxr.at[slot + 2]], ss, add=True)

    def wait_dma(gb, sem):
        pltpu.make_async_copy(xw.at[pl.ds(0, _CK)], gb, sem).wait()

    unpack(0, 0)
    gather(0, g0, sg0)
    unpack(1, 1)
    gather(1, g1, sg1)

    def pair(p, _):
        c0 = 2 * p
        c1 = c0 + 1
        wait_dma(g0, sg0)
        scale(c0, g0)
        scatter(0, g0, ss0)

        wait_dma(g1, sg1)
        scale(c1, g1)
        scatter(1, g1, ss1)

        wait_dma(g0, ss0)

        @pl.when(c0 + 2 < nch)
        def _():
            unpack(c0 + 2, 0)
            gather(0, g0, sg0)

        wait_dma(g1, ss1)

        @pl.when(c1 + 2 < nch)
        def _():
            unpack(c1 + 2, 1)
            gather(1, g1, sg1)
        return 0
    lax.fori_loop(0, nch // 2, pair, 0)
    plsc.subcore_barrier()

    def wout(k, _):
        pltpu.sync_copy(acc.at[pl.ds(s * _RPS + k * _CK, _CK)], g0)
        pltpu.sync_copy(g0, out.at[c, pl.ds(s * _RPS + k * _CK, _CK)])
        return 0
    lax.fori_loop(0, _RPS // _CK, wout, 0)


def _msg(xw, rc_r, norm_r):
    return pl.kernel(
        _msg_body,
        out_type=jax.ShapeDtypeStruct((2, _NP, _H), jnp.float32),
        mesh=_sc_mesh(),
        scratch_types=[
            pltpu.VMEM((_NCM, _CK), jnp.int32),
            pltpu.VMEM((_NCM, _CK), jnp.float32),
            pltpu.VMEM((4, _CK), jnp.int32),
            pltpu.VMEM((_CK, _H), jnp.float32),
            pltpu.VMEM((_CK, _H), jnp.float32),
            pltpu.VMEM_SHARED((_NP, _H), jnp.float32),
            pltpu.SemaphoreType.DMA,
            pltpu.SemaphoreType.DMA,
            pltpu.SemaphoreType.DMA,
            pltpu.SemaphoreType.DMA,
        ],
        compiler_params=_SC_PARAMS,
    )(xw, rc_r, norm_r)


def _dinv_body(degs, dinv, dinv2):
    d = degs[0:1, :] + degs[1:2, :] + 1.0
    r = lax.rsqrt(d)
    dinv[...] = r
    dinv2[...] = r * r


def _dinv_call(degs):
    return pl.pallas_call(
        _dinv_body,
        out_shape=[jax.ShapeDtypeStruct((1, _NP), jnp.float32)] * 2,
    )(degs)


def _mm1_body(x, w, o):
    o[...] = lax.dot_general(x[...], w[...], (((1,), (1,)), ((), ())),
                             preferred_element_type=jnp.float32)


def _mm1(x_p, w):
    return pl.pallas_call(
        _mm1_body,
        grid=(_NP // _BM,),
        in_specs=[
            pl.BlockSpec((_BM, _D), lambda i: (i, 0)),
            pl.BlockSpec((_H, _D), lambda i: (0, 0)),
        ],
        out_specs=pl.BlockSpec((_BM, _H), lambda i: (i, 0)),
        out_shape=jax.ShapeDtypeStruct((_NP, _H), jnp.float32),
    )(x_p, w)


def _fused_body(a0, a1, xw, d2, b, w, o):
    h = jnp.maximum(a0[...] + a1[...] + d2[...] * xw[...] + b[...], 0.0)
    o[...] = lax.dot_general(h, w[...], (((1,), (1,)), ((), ())),
                             preferred_element_type=jnp.float32)


def _fused(a0, a1, xw, d2c, br, w):
    return pl.pallas_call(
        _fused_body,
        grid=(_NP // _BM,),
        in_specs=[
            pl.BlockSpec((_BM, _H), lambda i: (i, 0)),
            pl.BlockSpec((_BM, _H), lambda i: (i, 0)),
            pl.BlockSpec((_BM, _H), lambda i: (i, 0)),
            pl.BlockSpec((_BM, 1), lambda i: (i, 0)),
            pl.BlockSpec((1, _H), lambda i: (0, 0)),
            pl.BlockSpec((_H, _H), lambda i: (0, 0)),
        ],
        out_specs=pl.BlockSpec((_BM, _H), lambda i: (i, 0)),
        out_shape=jax.ShapeDtypeStruct((_NP, _H), jnp.float32),
    )(a0, a1, xw, d2c, br, w)


def _final_body(a0, a1, xw, d2, b, batch, wl, bl, o, sums, cnt):
    i = pl.program_id(0)

    @pl.when(i == 0)
    def _():
        sums[...] = jnp.zeros_like(sums)
        cnt[...] = jnp.zeros_like(cnt)

    h = a0[...] + a1[...] + d2[...] * xw[...] + b[...]
    gi = lax.broadcasted_iota(jnp.int32, (_G, _BM), 0)
    m = (gi == batch[...]).astype(jnp.float32)
    sums[...] += lax.dot_general(m, h, (((1,), (0,)), ((), ())),
                                 preferred_element_type=jnp.float32)
    cnt[...] += jnp.sum(m, axis=1, keepdims=True)

    @pl.when(i == _NP // _BM - 1)
    def _():
        mean = sums[...] / jnp.maximum(cnt[...], 1.0)
        o[...] = lax.dot_general(mean, wl[...], (((1,), (1,)), ((), ())),
                                 preferred_element_type=jnp.float32) + bl[...]


def _final(a0, a1, xw, d2c, br, batch_p, wl, blr):
    return pl.pallas_call(
        _final_body,
        grid=(_NP // _BM,),
        in_specs=[
            pl.BlockSpec((_BM, _H), lambda i: (i, 0)),
            pl.BlockSpec((_BM, _H), lambda i: (i, 0)),
            pl.BlockSpec((_BM, _H), lambda i: (i, 0)),
            pl.BlockSpec((_BM, 1), lambda i: (i, 0)),
            pl.BlockSpec((1, _H), lambda i: (0, 0)),
            pl.BlockSpec((1, _BM), lambda i: (0, i)),
            pl.BlockSpec((2, _H), lambda i: (0, 0)),
            pl.BlockSpec((1, 2), lambda i: (0, 0)),
        ],
        out_specs=pl.BlockSpec((_G, 2), lambda i: (0, 0)),
        out_shape=jax.ShapeDtypeStruct((_G, 2), jnp.float32),
        scratch_shapes=[
            pltpu.VMEM((_G, _H), jnp.float32),
            pltpu.VMEM((_G, 1), jnp.float32),
        ],
    )(a0, a1, xw, d2c, br, batch_p, wl, blr)


def kernel(x, edge_index, edge_attr, batch, W1, b1, W2, b2, W3, b3, Wl, bl):
    pe = _EPAD - _E

    def _part(a):
        a0 = a[: 16 * _NC0 * _CK].reshape(16, _NC0, _CK)
        a1 = a[16 * _NC0 * _CK:].reshape(16, _NC1, _CK)
        o = jnp.zeros((_NT, _NCM, _CK), a.dtype)
        return o.at[:16, :_NC0].set(a0).at[16:, :_NC1].set(a1)

    row_r = _part(jnp.pad(edge_index[0], (0, pe)))
    col_r = _part(jnp.pad(edge_index[1], (0, pe)))
    w_r = _part(jnp.pad(edge_attr, (0, pe)))
    x_p = jnp.pad(x, ((0, _NP - _N), (0, 0)))
    batch_p = jnp.pad(batch, (0, _NP - _N), constant_values=_G).reshape(1, _NP)
    b1r = b1.reshape(1, _H)
    b2r = b2.reshape(1, _H)
    b3r = b3.reshape(1, _H)
    blr = bl.reshape(1, 2)

    degs = _deg(col_r, w_r)
    dinv, dinv2 = _dinv_call(degs)
    d2c = dinv2.reshape(_NP, 1)
    norm_r, rc_r = _normk(row_r, col_r, w_r, dinv)

    xw1 = _mm1(x_p, W1)
    acc = _msg(xw1, rc_r, norm_r)
    xw2 = _fused(acc[0], acc[1], xw1, d2c, b1r, W2)
    acc = _msg(xw2, rc_r, norm_r)
    xw3 = _fused(acc[0], acc[1], xw2, d2c, b2r, W3)
    acc = _msg(xw3, rc_r, norm_r)
    return _final(acc[0], acc[1], xw3, d2c, b3r, batch_p, Wl, blr)


# speedup vs baseline: 1.2634x; 1.1951x over previous
import jax
import jax.numpy as jnp
from jax import lax
from jax.experimental import pallas as pl
from jax.experimental.pallas import tpu as pltpu
from jax.experimental.pallas import tpu_sc as plsc

_N = 10000
_E = 160000
_D = 384
_H = 128
_G = 64
_NP = 10240
_EPAD = 163840
_NT = 32
_NC = 80
_CK = 64
_RPS = _NP // 16
_BM = 1024


def _sc_mesh():
    return plsc.VectorSubcoreMesh(core_axis_name="c", subcore_axis_name="s")


_SC_PARAMS = pltpu.CompilerParams(needs_layout_passes=False)


def _deg_body(col_r, w_r, degs, colb, wb, zv, stage, dacc):
    c = lax.axis_index("c")
    s = lax.axis_index("s")
    wid = c * 16 + s
    pltpu.sync_copy(col_r.at[wid], colb)
    pltpu.sync_copy(w_r.at[wid], wb)

    def z(i, _):
        zv[pl.ds(i * 16, 16)] = jnp.zeros((16,), jnp.float32)
        return 0
    lax.fori_loop(0, _RPS // 16, z, 0)
    pltpu.sync_copy(zv, dacc.at[pl.ds(s * _RPS, _RPS)])
    plsc.subcore_barrier()

    def step(ci, _):
        pltpu.sync_copy(wb.at[ci], dacc.at[colb.at[ci]], add=True)
        return 0
    lax.fori_loop(0, _NC, step, 0)
    plsc.subcore_barrier()

    pltpu.sync_copy(dacc.at[pl.ds(s * _RPS, _RPS)], stage)
    pltpu.sync_copy(stage, degs.at[c, pl.ds(s * _RPS, _RPS)])


def _deg(col_r, w_r):
    return pl.kernel(
        _deg_body,
        out_type=jax.ShapeDtypeStruct((2, _NP), jnp.float32),
        mesh=_sc_mesh(),
        scratch_types=[
            pltpu.VMEM((_NC, _CK), jnp.int32),
            pltpu.VMEM((_NC, _CK), jnp.float32),
            pltpu.VMEM((_RPS,), jnp.float32),
            pltpu.VMEM((_RPS,), jnp.float32),
            pltpu.VMEM_SHARED((_NP,), jnp.float32),
        ],
        compiler_params=_SC_PARAMS,
    )(col_r, w_r)


def _norm_body(row_r, col_r, w_r, dinv, out, rowb, colb, wb, dv):
    c = lax.axis_index("c")
    s = lax.axis_index("s")
    wid = c * 16 + s
    pltpu.sync_copy(row_r.at[wid], rowb)
    pltpu.sync_copy(col_r.at[wid], colb)
    pltpu.sync_copy(w_r.at[wid], wb)
    pltpu.sync_copy(dinv.at[0], dv)

    def ngroup(g, _):
        ci = g // (_CK // 16)
        off = (g % (_CK // 16)) * 16
        rv = rowb[ci, pl.ds(off, 16)]
        cv = colb[ci, pl.ds(off, 16)]
        wv = wb[ci, pl.ds(off, 16)]
        wb[ci, pl.ds(off, 16)] = (
            plsc.load_gather(dv, [rv]) * wv * plsc.load_gather(dv, [cv]))
        return 0
    lax.fori_loop(0, _NC * (_CK // 16), ngroup, 0)
    pltpu.sync_copy(wb, out.at[wid])


def _normk(row_r, col_r, w_r, dinv):
    return pl.kernel(
        _norm_body,
        out_type=jax.ShapeDtypeStruct((_NT, _NC, _CK), jnp.float32),
        mesh=_sc_mesh(),
        scratch_types=[
            pltpu.VMEM((_NC, _CK), jnp.int32),
            pltpu.VMEM((_NC, _CK), jnp.int32),
            pltpu.VMEM((_NC, _CK), jnp.float32),
            pltpu.VMEM((_NP,), jnp.float32),
        ],
        compiler_params=_SC_PARAMS,
    )(row_r, col_r, w_r, dinv)


def _msg_body(xw, row_r, col_r, norm_r, out, rowb, colb, nb,
              g0, g1, acc, sg0, sg1, ss0, ss1):
    c = lax.axis_index("c")
    s = lax.axis_index("s")
    wid = c * 16 + s
    pltpu.sync_copy(row_r.at[wid], rowb)
    pltpu.sync_copy(col_r.at[wid], colb)
    pltpu.sync_copy(norm_r.at[wid], nb)

    def z(i, _):
        for j in range(_H // 16):
            g0[i, pl.ds(j * 16, 16)] = jnp.zeros((16,), jnp.float32)
        return 0
    lax.fori_loop(0, _CK, z, 0)

    def zrow(k, _):
        pltpu.sync_copy(g0, acc.at[pl.ds(s * _RPS + k * _CK, _CK)])
        return 0
    lax.fori_loop(0, _RPS // _CK, zrow, 0)
    plsc.subcore_barrier()

    def gather(ci, gb, sg):
        pltpu.async_copy(xw.at[rowb.at[ci]], gb, sg)

    def scale(ci, gb, sb):
        def group(g, _):
            nv = nb[ci, pl.ds(g * 16, 16)]
            base = g * 16
            for j in range(16):
                nbr = jnp.full((16,), nv[j])
                for k in range(_H // 16):
                    sl = pl.ds(k * 16, 16)
                    sb[base + j, sl] = gb[base + j, sl] * nbr
            return 0
        lax.fori_loop(0, _CK // 16, group, 0)

    def scatter(ci, sb, ss):
        pltpu.async_copy(sb, acc.at[colb.at[ci]], ss, add=True)

    def wait_gather(ci, gb, sg):
        pltpu.make_async_copy(xw.at[rowb.at[ci]], gb, sg).wait()

    def wait_scatter(ci, sb, ss):
        pltpu.make_async_copy(sb, acc.at[colb.at[ci]], ss).wait()

    gather(0, g0, sg0)
    gather(1, g1, sg1)

    def pair(p, _):
        c0 = 2 * p
        c1 = c0 + 1
        wait_gather(c0, g0, sg0)
        scale(c0, g0, g0)
        scatter(c0, g0, ss0)

        wait_gather(c1, g1, sg1)
        scale(c1, g1, g1)
        scatter(c1, g1, ss1)

        wait_scatter(c0, g0, ss0)

        @pl.when(c0 + 2 < _NC)
        def _():
            gather(c0 + 2, g0, sg0)

        wait_scatter(c1, g1, ss1)

        @pl.when(c1 + 2 < _NC)
        def _():
            gather(c1 + 2, g1, sg1)
        return 0
    lax.fori_loop(0, _NC // 2, pair, 0)
    plsc.subcore_barrier()

    def wout(k, _):
        pltpu.sync_copy(acc.at[pl.ds(s * _RPS + k * _CK, _CK)], g0)
        pltpu.sync_copy(g0, out.at[c, pl.ds(s * _RPS + k * _CK, _CK)])
        return 0
    lax.fori_loop(0, _RPS // _CK, wout, 0)


def _msg(xw, row_r, col_r, norm_r):
    return pl.kernel(
        _msg_body,
        out_type=jax.ShapeDtypeStruct((2, _NP, _H), jnp.float32),
        mesh=_sc_mesh(),
        scratch_types=[
            pltpu.VMEM((_NC, _CK), jnp.int32),
            pltpu.VMEM((_NC, _CK), jnp.int32),
            pltpu.VMEM((_NC, _CK), jnp.float32),
            pltpu.VMEM((_CK, _H), jnp.float32),
            pltpu.VMEM((_CK, _H), jnp.float32),
            pltpu.VMEM_SHARED((_NP, _H), jnp.float32),
            pltpu.SemaphoreType.DMA,
            pltpu.SemaphoreType.DMA,
            pltpu.SemaphoreType.DMA,
            pltpu.SemaphoreType.DMA,
        ],
        compiler_params=_SC_PARAMS,
    )(xw, row_r, col_r, norm_r)


def _dinv_body(degs, dinv, dinv2):
    d = degs[0:1, :] + degs[1:2, :] + 1.0
    r = lax.rsqrt(d)
    dinv[...] = r
    dinv2[...] = r * r


def _dinv_call(degs):
    return pl.pallas_call(
        _dinv_body,
        out_shape=[jax.ShapeDtypeStruct((1, _NP), jnp.float32)] * 2,
    )(degs)


def _mm1_body(x, w, o):
    o[...] = lax.dot_general(x[...], w[...], (((1,), (1,)), ((), ())),
                             preferred_element_type=jnp.float32)


def _mm1(x_p, w):
    return pl.pallas_call(
        _mm1_body,
        grid=(_NP // _BM,),
        in_specs=[
            pl.BlockSpec((_BM, _D), lambda i: (i, 0)),
            pl.BlockSpec((_H, _D), lambda i: (0, 0)),
        ],
        out_specs=pl.BlockSpec((_BM, _H), lambda i: (i, 0)),
        out_shape=jax.ShapeDtypeStruct((_NP, _H), jnp.float32),
    )(x_p, w)


def _fused_body(a0, a1, xw, d2, b, w, o):
    h = jnp.maximum(a0[...] + a1[...] + d2[...] * xw[...] + b[...], 0.0)
    o[...] = lax.dot_general(h, w[...], (((1,), (1,)), ((), ())),
                             preferred_element_type=jnp.float32)


def _fused(a0, a1, xw, d2c, br, w):
    return pl.pallas_call(
        _fused_body,
        grid=(_NP // _BM,),
        in_specs=[
            pl.BlockSpec((_BM, _H), lambda i: (i, 0)),
            pl.BlockSpec((_BM, _H), lambda i: (i, 0)),
            pl.BlockSpec((_BM, _H), lambda i: (i, 0)),
            pl.BlockSpec((_BM, 1), lambda i: (i, 0)),
            pl.BlockSpec((1, _H), lambda i: (0, 0)),
            pl.BlockSpec((_H, _H), lambda i: (0, 0)),
        ],
        out_specs=pl.BlockSpec((_BM, _H), lambda i: (i, 0)),
        out_shape=jax.ShapeDtypeStruct((_NP, _H), jnp.float32),
    )(a0, a1, xw, d2c, br, w)


def _final_body(a0, a1, xw, d2, b, batch, wl, bl, o, sums, cnt):
    i = pl.program_id(0)

    @pl.when(i == 0)
    def _():
        sums[...] = jnp.zeros_like(sums)
        cnt[...] = jnp.zeros_like(cnt)

    h = a0[...] + a1[...] + d2[...] * xw[...] + b[...]
    gi = lax.broadcasted_iota(jnp.int32, (_G, _BM), 0)
    m = (gi == batch[...]).astype(jnp.float32)
    sums[...] += lax.dot_general(m, h, (((1,), (0,)), ((), ())),
                                 preferred_element_type=jnp.float32)
    cnt[...] += jnp.sum(m, axis=1, keepdims=True)

    @pl.when(i == _NP // _BM - 1)
    def _():
        mean = sums[...] / jnp.maximum(cnt[...], 1.0)
        o[...] = lax.dot_general(mean, wl[...], (((1,), (1,)), ((), ())),
                                 preferred_element_type=jnp.float32) + bl[...]


def _final(a0, a1, xw, d2c, br, batch_p, wl, blr):
    return pl.pallas_call(
        _final_body,
        grid=(_NP // _BM,),
        in_specs=[
            pl.BlockSpec((_BM, _H), lambda i: (i, 0)),
            pl.BlockSpec((_BM, _H), lambda i: (i, 0)),
            pl.BlockSpec((_BM, _H), lambda i: (i, 0)),
            pl.BlockSpec((_BM, 1), lambda i: (i, 0)),
            pl.BlockSpec((1, _H), lambda i: (0, 0)),
            pl.BlockSpec((1, _BM), lambda i: (0, i)),
            pl.BlockSpec((2, _H), lambda i: (0, 0)),
            pl.BlockSpec((1, 2), lambda i: (0, 0)),
        ],
        out_specs=pl.BlockSpec((_G, 2), lambda i: (0, 0)),
        out_shape=jax.ShapeDtypeStruct((_G, 2), jnp.float32),
        scratch_shapes=[
            pltpu.VMEM((_G, _H), jnp.float32),
            pltpu.VMEM((_G, 1), jnp.float32),
        ],
    )(a0, a1, xw, d2c, br, batch_p, wl, blr)


def kernel(x, edge_index, edge_attr, batch, W1, b1, W2, b2, W3, b3, Wl, bl):
    pe = _EPAD - _E
    row_r = jnp.pad(edge_index[0], (0, pe)).reshape(_NT, _NC, _CK)
    col_r = jnp.pad(edge_index[1], (0, pe)).reshape(_NT, _NC, _CK)
    w_r = jnp.pad(edge_attr, (0, pe)).reshape(_NT, _NC, _CK)
    x_p = jnp.pad(x, ((0, _NP - _N), (0, 0)))
    batch_p = jnp.pad(batch, (0, _NP - _N), constant_values=_G).reshape(1, _NP)
    b1r = b1.reshape(1, _H)
    b2r = b2.reshape(1, _H)
    b3r = b3.reshape(1, _H)
    blr = bl.reshape(1, 2)

    degs = _deg(col_r, w_r)
    dinv, dinv2 = _dinv_call(degs)
    d2c = dinv2.reshape(_NP, 1)
    norm_r = _normk(row_r, col_r, w_r, dinv)

    xw1 = _mm1(x_p, W1)
    acc = _msg(xw1, row_r, col_r, norm_r)
    xw2 = _fused(acc[0], acc[1], xw1, d2c, b1r, W2)
    acc = _msg(xw2, row_r, col_r, norm_r)
    xw3 = _fused(acc[0], acc[1], xw2, d2c, b2r, W3)
    acc = _msg(xw3, row_r, col_r, norm_r)
    return _final(acc[0], acc[1], xw3, d2c, b3r, batch_p, Wl, blr)
